# initial kernel scaffold (unmeasured)
import functools

import jax
import jax.numpy as jnp
from jax import lax
from jax.experimental import pallas as pl
from jax.experimental.pallas import tpu as pltpu

N_DEV = 4
SQ = 1024
SQ_SHARD = 256
SKV = 4096
HQ_LOCAL = 8
DH = 128
D_LOCAL = HQ_LOCAL * DH
D_MODEL = 1024
SCALE = 0.08838834764831843
BLK = 64
RC = 256
N_RC = SQ // RC


def kernel(x, Wq, K_ext, V_ext, Wo):
    def body(x_ref, wq_ref, k_hbm, v_hbm, wo_ref, out_ref,
             x_all, q_bf, k_bf, v_bf, kst, vst, p_ref, rs_send, rs_recv,
             ctx_bf, ag_ssem, ag_rsem, rs_ssem, rs_rsem, ld_sem):
        my = lax.axis_index("i")
        right = lax.rem(my + 1, N_DEV)

        xv = x_ref[0].astype(jnp.bfloat16)
        pl.store(x_all, (pl.ds(my, 1), slice(None), slice(None)), xv[None])
        for h in range(N_DEV - 1):
            sl = lax.rem(my - h + N_DEV, N_DEV)
            rdma = pltpu.make_async_remote_copy(
                src_ref=x_all.at[sl],
                dst_ref=x_all.at[sl],
                send_sem=ag_ssem.at[h],
                recv_sem=ag_rsem.at[h],
                device_id=(right,),
                device_id_type=pl.DeviceIdType.MESH,
            )
            rdma.start()
            rdma.wait()

        for h in range(HQ_LOCAL):
            hk = my * HQ_LOCAL + h
            cpk = pltpu.make_async_copy(
                k_hbm.at[0, :, hk, :], kst, ld_sem.at[0])
            cpv = pltpu.make_async_copy(
                v_hbm.at[0, :, hk, :], vst, ld_sem.at[1])
            cpk.start()
            cpv.start()
            cpk.wait()
            cpv.wait()
            k_bf[h] = kst[...].astype(jnp.bfloat16)
            v_bf[h] = vst[...].astype(jnp.bfloat16)

        xg = x_all[...].reshape(SQ, D_MODEL)
        wq_b = wq_ref[...].astype(jnp.bfloat16)
        qf = jax.lax.dot_general(
            xg, wq_b, (((1,), (0,)), ((), ())),
            preferred_element_type=jnp.float32)
        q_bf[...] = qf.astype(jnp.bfloat16)

        for rc in range(N_RC):
            r0 = rc * RC
            qb = (lax.broadcasted_iota(jnp.int32, (RC, SKV), 0) + r0) // BLK
            kb = lax.broadcasted_iota(jnp.int32, (RC, SKV), 1) // BLK
            allowed = (qb == kb) | (kb == 0) | (((qb + kb) % 3) == 0)
            bias = jnp.where(allowed, 0.0, -1e9).astype(jnp.float32)
            for h in range(HQ_LOCAL):
                q = q_bf[r0:r0 + RC, h * DH:(h + 1) * DH]
                s = jax.lax.dot_general(
                    q, k_bf[h], (((1,), (1,)), ((), ())),
                    preferred_element_type=jnp.float32)
                s = s * SCALE + bias
                m = jnp.max(s, axis=1, keepdims=True)
                w = jnp.exp(s - m)
                den = jnp.sum(w, axis=1, keepdims=True)
                wb = (w / den).astype(jnp.bfloat16)
                ctx = jax.lax.dot_general(
                    wb, v_bf[h], (((1,), (0,)), ((), ())),
                    preferred_element_type=jnp.float32)
                ctx_bf[r0:r0 + RC, h * DH:(h + 1) * DH] = ctx.astype(
                    jnp.bfloat16)

        wo_b = wo_ref[...].astype(jnp.bfloat16)
        pf = jax.lax.dot_general(
            ctx_bf[...], wo_b, (((1,), (0,)), ((), ())),
            preferred_element_type=jnp.float32)
        p_ref[...] = pf.reshape(N_DEV, SQ_SHARD, D_MODEL)

        acc = None
        for t in range(N_DEV - 1):
            cs = lax.rem(my - 1 - t + 2 * N_DEV, N_DEV)
            cr = lax.rem(my - 2 - t + 2 * N_DEV, N_DEV)
            if t == 0:
                sendv = pl.load(
                    p_ref, (pl.ds(cs, 1), slice(None), slice(None))
                ).reshape(SQ_SHARD, D_MODEL)
            else:
                sendv = acc
            rs_send[t] = sendv.astype(jnp.bfloat16)
            rdma = pltpu.make_async_remote_copy(
                src_ref=rs_send.at[t],
                dst_ref=rs_recv.at[t],
                send_sem=rs_ssem.at[t],
                recv_sem=rs_rsem.at[t],
                device_id=(right,),
                device_id_type=pl.DeviceIdType.MESH,
            )
            rdma.start()
            rdma.wait()
            ploc = pl.load(
                p_ref, (pl.ds(cr, 1), slice(None), slice(None))
            ).reshape(SQ_SHARD, D_MODEL)
            acc = rs_recv[t].astype(jnp.float32) + ploc
        out_ref[0] = acc

    out_shape = jax.ShapeDtypeStruct((1, SQ_SHARD, D_MODEL), jnp.float32)
    return pl.pallas_call(
        body,
        out_shape=out_shape,
        in_specs=[
            pl.BlockSpec(memory_space=pltpu.VMEM),
            pl.BlockSpec(memory_space=pltpu.VMEM),
            pl.BlockSpec(memory_space=pltpu.ANY),
            pl.BlockSpec(memory_space=pltpu.ANY),
            pl.BlockSpec(memory_space=pltpu.VMEM),
        ],
        out_specs=pl.BlockSpec(memory_space=pltpu.VMEM),
        scratch_shapes=[
            pltpu.VMEM((N_DEV, SQ_SHARD, D_MODEL), jnp.bfloat16),
            pltpu.VMEM((SQ, D_LOCAL), jnp.bfloat16),
            pltpu.VMEM((HQ_LOCAL, SKV, DH), jnp.bfloat16),
            pltpu.VMEM((HQ_LOCAL, SKV, DH), jnp.bfloat16),
            pltpu.VMEM((SKV, DH), jnp.float32),
            pltpu.VMEM((SKV, DH), jnp.float32),
            pltpu.VMEM((N_DEV, SQ_SHARD, D_MODEL), jnp.float32),
            pltpu.VMEM((N_DEV - 1, SQ_SHARD, D_MODEL), jnp.bfloat16),
            pltpu.VMEM((N_DEV - 1, SQ_SHARD, D_MODEL), jnp.bfloat16),
            pltpu.VMEM((SQ, D_LOCAL), jnp.bfloat16),
            pltpu.SemaphoreType.DMA((N_DEV - 1,)),
            pltpu.SemaphoreType.DMA((N_DEV - 1,)),
            pltpu.SemaphoreType.DMA((N_DEV - 1,)),
            pltpu.SemaphoreType.DMA((N_DEV - 1,)),
            pltpu.SemaphoreType.DMA((2,)),
        ],
        compiler_params=pltpu.CompilerParams(collective_id=0),
    )(x, Wq, K_ext, V_ext, Wo)


# baseline (device time: 159253 ns/iter reference)
import functools

import jax
import jax.numpy as jnp
from jax import lax
from jax.experimental import pallas as pl
from jax.experimental.pallas import tpu as pltpu

N_DEV = 4
SQ = 1024
SQ_SHARD = 256
SKV = 4096
HQ_LOCAL = 8
DH = 128
D_LOCAL = HQ_LOCAL * DH
D_MODEL = 1024
SCALE = 0.08838834764831843
BLK = 64
RC = 256
N_RC = SQ // RC


def kernel(x, Wq, K_ext, V_ext, Wo):
    def body(x_ref, wq_ref, k_hbm, v_hbm, wo_ref, out_ref,
             x_all, q_bf, k_bf, v_bf, kst, vst, p_ref, rs_send, rs_recv,
             ctx_bf, ag_ssem, ag_rsem, rs_ssem, rs_rsem, ld_sem):
        my = lax.axis_index("i")
        right = lax.rem(my + 1, N_DEV)

        xv = x_ref[0].astype(jnp.bfloat16)
        x_all[pl.ds(my, 1)] = xv[None]
        for h in range(N_DEV - 1):
            sl = lax.rem(my - h + N_DEV, N_DEV)
            rdma = pltpu.make_async_remote_copy(
                src_ref=x_all.at[sl],
                dst_ref=x_all.at[sl],
                send_sem=ag_ssem.at[h],
                recv_sem=ag_rsem.at[h],
                device_id=(right,),
                device_id_type=pl.DeviceIdType.MESH,
            )
            rdma.start()
            rdma.wait()

        for h in range(HQ_LOCAL):
            hk = my * HQ_LOCAL + h
            cpk = pltpu.make_async_copy(
                k_hbm.at[0, :, hk, :], kst, ld_sem.at[0])
            cpv = pltpu.make_async_copy(
                v_hbm.at[0, :, hk, :], vst, ld_sem.at[1])
            cpk.start()
            cpv.start()
            cpk.wait()
            cpv.wait()
            k_bf[h] = kst[...].astype(jnp.bfloat16)
            v_bf[h] = vst[...].astype(jnp.bfloat16)

        xg = x_all[...].reshape(SQ, D_MODEL)
        wq_b = wq_ref[...].astype(jnp.bfloat16)
        qf = jax.lax.dot_general(
            xg, wq_b, (((1,), (0,)), ((), ())),
            preferred_element_type=jnp.float32)
        q_bf[...] = qf.astype(jnp.bfloat16)

        for rc in range(N_RC):
            r0 = rc * RC
            qb = (lax.broadcasted_iota(jnp.int32, (RC, SKV), 0) + r0) // BLK
            kb = lax.broadcasted_iota(jnp.int32, (RC, SKV), 1) // BLK
            allowed = (qb == kb) | (kb == 0) | (((qb + kb) % 3) == 0)
            bias = jnp.where(allowed, 0.0, -1e9).astype(jnp.float32)
            for h in range(HQ_LOCAL):
                q = q_bf[r0:r0 + RC, h * DH:(h + 1) * DH]
                s = jax.lax.dot_general(
                    q, k_bf[h], (((1,), (1,)), ((), ())),
                    preferred_element_type=jnp.float32)
                s = s * SCALE + bias
                m = jnp.max(s, axis=1, keepdims=True)
                w = jnp.exp(s - m)
                den = jnp.sum(w, axis=1, keepdims=True)
                wb = (w / den).astype(jnp.bfloat16)
                ctx = jax.lax.dot_general(
                    wb, v_bf[h], (((1,), (0,)), ((), ())),
                    preferred_element_type=jnp.float32)
                ctx_bf[r0:r0 + RC, h * DH:(h + 1) * DH] = ctx.astype(
                    jnp.bfloat16)

        wo_b = wo_ref[...].astype(jnp.bfloat16)
        pf = jax.lax.dot_general(
            ctx_bf[...], wo_b, (((1,), (0,)), ((), ())),
            preferred_element_type=jnp.float32)
        p_ref[...] = pf.reshape(N_DEV, SQ_SHARD, D_MODEL)

        acc = None
        for t in range(N_DEV - 1):
            cs = lax.rem(my - 1 - t + 2 * N_DEV, N_DEV)
            cr = lax.rem(my - 2 - t + 2 * N_DEV, N_DEV)
            if t == 0:
                sendv = p_ref[pl.ds(cs, 1)].reshape(SQ_SHARD, D_MODEL)
            else:
                sendv = acc
            rs_send[t] = sendv.astype(jnp.bfloat16)
            rdma = pltpu.make_async_remote_copy(
                src_ref=rs_send.at[t],
                dst_ref=rs_recv.at[t],
                send_sem=rs_ssem.at[t],
                recv_sem=rs_rsem.at[t],
                device_id=(right,),
                device_id_type=pl.DeviceIdType.MESH,
            )
            rdma.start()
            rdma.wait()
            ploc = p_ref[pl.ds(cr, 1)].reshape(SQ_SHARD, D_MODEL)
            acc = rs_recv[t].astype(jnp.float32) + ploc
        out_ref[0] = acc

    out_shape = jax.ShapeDtypeStruct((1, SQ_SHARD, D_MODEL), jnp.float32)
    return pl.pallas_call(
        body,
        out_shape=out_shape,
        in_specs=[
            pl.BlockSpec(memory_space=pltpu.VMEM),
            pl.BlockSpec(memory_space=pltpu.VMEM),
            pl.BlockSpec(memory_space=pl.ANY),
            pl.BlockSpec(memory_space=pl.ANY),
            pl.BlockSpec(memory_space=pltpu.VMEM),
        ],
        out_specs=pl.BlockSpec(memory_space=pltpu.VMEM),
        scratch_shapes=[
            pltpu.VMEM((N_DEV, SQ_SHARD, D_MODEL), jnp.bfloat16),
            pltpu.VMEM((SQ, D_LOCAL), jnp.bfloat16),
            pltpu.VMEM((HQ_LOCAL, SKV, DH), jnp.bfloat16),
            pltpu.VMEM((HQ_LOCAL, SKV, DH), jnp.bfloat16),
            pltpu.VMEM((SKV, DH), jnp.float32),
            pltpu.VMEM((SKV, DH), jnp.float32),
            pltpu.VMEM((N_DEV, SQ_SHARD, D_MODEL), jnp.float32),
            pltpu.VMEM((N_DEV - 1, SQ_SHARD, D_MODEL), jnp.bfloat16),
            pltpu.VMEM((N_DEV - 1, SQ_SHARD, D_MODEL), jnp.bfloat16),
            pltpu.VMEM((SQ, D_LOCAL), jnp.bfloat16),
            pltpu.SemaphoreType.DMA((N_DEV - 1,)),
            pltpu.SemaphoreType.DMA((N_DEV - 1,)),
            pltpu.SemaphoreType.DMA((N_DEV - 1,)),
            pltpu.SemaphoreType.DMA((N_DEV - 1,)),
            pltpu.SemaphoreType.DMA((2,)),
        ],
        compiler_params=pltpu.CompilerParams(
            vmem_limit_bytes=100 * 1024 * 1024),
    )(x, Wq, K_ext, V_ext, Wo)


# device time: 107828 ns/iter; 1.4769x vs baseline; 1.4769x over previous
import jax
import jax.numpy as jnp
from jax import lax
from jax.experimental import pallas as pl
from jax.experimental.pallas import tpu as pltpu

N_DEV = 4
SQ = 1024
SQ_SHARD = 256
SKV = 4096
HQ_LOCAL = 8
DH = 128
D_LOCAL = HQ_LOCAL * DH
D_MODEL = 1024
SCALE = 0.08838834764831843
BLK = 64
N_QB = SQ // BLK
N_KB = SKV // BLK

SORTED_KB = (list(range(0, N_KB, 3)) + list(range(1, N_KB, 3))
             + list(range(2, N_KB, 3)))
POS = {kb: j for j, kb in enumerate(SORTED_KB)}
CNT = [len(range(c, N_KB, 3)) for c in range(3)]
OFF = [0, CNT[0], CNT[0] + CNT[1]]

QBS_G = [[qb for qb in range(N_QB) if qb % 3 == g] for g in range(3)]
ROW_OFF = [0, len(QBS_G[0]) * BLK,
           (len(QBS_G[0]) + len(QBS_G[1])) * BLK]
SROW = QBS_G[0] + QBS_G[1] + QBS_G[2]
NEX = 1 + len(QBS_G[1])


def kernel(x, Wq, K_ext, V_ext, Wo):
    def body(x_ref, wq_ref, k_hbm, v_hbm, wo_ref, out_ref,
             x_all, q_bf, k_bf, v_bf, kst, vst, p_ref, rs_send, rs_recv,
             ctx_bf, ag_ssem, ag_rsem, rs_ssem, rs_rsem, ld_sem):
        my = lax.axis_index("i")
        right = lax.rem(my + 1, N_DEV)

        def start_head(h):
            s = h % 2
            hk = my * HQ_LOCAL + h
            ds = []
            for j, kb in enumerate(SORTED_KB):
                ck = pltpu.make_async_copy(
                    k_hbm.at[0, pl.ds(kb * BLK, BLK), hk, :],
                    kst.at[s, pl.ds(j * BLK, BLK), :],
                    ld_sem.at[s, 0])
                cv = pltpu.make_async_copy(
                    v_hbm.at[0, pl.ds(kb * BLK, BLK), hk, :],
                    vst.at[s, pl.ds(j * BLK, BLK), :],
                    ld_sem.at[s, 1])
                ck.start()
                cv.start()
                ds += [ck, cv]
            return ds

        descs = [start_head(0), start_head(1)]

        xv = x_ref[0].astype(jnp.bfloat16)
        x_all[pl.ds(my, 1)] = xv[None]
        for h in range(N_DEV - 1):
            sl = lax.rem(my - h + N_DEV, N_DEV)
            rdma = pltpu.make_async_remote_copy(
                src_ref=x_all.at[sl],
                dst_ref=x_all.at[sl],
                send_sem=ag_ssem.at[h],
                recv_sem=ag_rsem.at[h],
                device_id=(right,),
                device_id_type=pl.DeviceIdType.MESH,
            )
            rdma.start()
            rdma.wait()

        xg = x_all[...].reshape(SQ, D_MODEL)
        xs = jnp.concatenate(
            [xg[qb * BLK:(qb + 1) * BLK] for qb in SROW], axis=0)
        wq_b = wq_ref[...].astype(jnp.bfloat16)
        for half in range(2):
            qf = jax.lax.dot_general(
                xs[half * 512:(half + 1) * 512], wq_b,
                (((1,), (0,)), ((), ())),
                preferred_element_type=jnp.float32)
            q_bf[half * 512:(half + 1) * 512] = qf.astype(jnp.bfloat16)

        for h in range(HQ_LOCAL):
            for d in descs[h]:
                d.wait()
            s = h % 2
            k_bf[h] = kst[s].astype(jnp.bfloat16)
            v_bf[h] = vst[s].astype(jnp.bfloat16)
            if h + 2 < HQ_LOCAL:
                descs.append(start_head(h + 2))

        rb = lax.broadcasted_iota(jnp.int32, (ROW_OFF[2] - ROW_OFF[1],
                                              NEX * BLK), 0) // BLK
        cb = lax.broadcasted_iota(jnp.int32, (ROW_OFF[2] - ROW_OFF[1],
                                              NEX * BLK), 1) // BLK
        bias_ex = jnp.where((cb == 0) | (rb == cb - 1), 0.0,
                            -1e9).astype(jnp.float32)
        for g in range(3):
            r = (3 - g) % 3
            roff = ROW_OFF[g]
            m_rows = len(QBS_G[g]) * BLK
            c0 = OFF[r] * BLK
            cw = CNT[r] * BLK
            for h in range(HQ_LOCAL):
                q = q_bf[roff:roff + m_rows, h * DH:(h + 1) * DH]
                s_main = jax.lax.dot_general(
                    q, k_bf[h, c0:c0 + cw, :], (((1,), (1,)), ((), ())),
                    preferred_element_type=jnp.float32) * SCALE
                if g == 0:
                    m = jnp.max(s_main, axis=1, keepdims=True)
                    w = jnp.exp(s_main - m)
                    den = jnp.sum(w, axis=1, keepdims=True)
                    wb = (w / den).astype(jnp.bfloat16)
                    ctx = jax.lax.dot_general(
                        wb, v_bf[h, c0:c0 + cw, :], (((1,), (0,)), ((), ())),
                        preferred_element_type=jnp.float32)
                else:
                    k_ex = jnp.concatenate(
                        [k_bf[h, 0:BLK, :]]
                        + [k_bf[h, POS[qb] * BLK:(POS[qb] + 1) * BLK, :]
                           for qb in QBS_G[g]], axis=0)
                    v_ex = jnp.concatenate(
                        [v_bf[h, 0:BLK, :]]
                        + [v_bf[h, POS[qb] * BLK:(POS[qb] + 1) * BLK, :]
                           for qb in QBS_G[g]], axis=0)
                    s_ex = jax.lax.dot_general(
                        q, k_ex, (((1,), (1,)), ((), ())),
                        preferred_element_type=jnp.float32) * SCALE + bias_ex
                    m = jnp.maximum(jnp.max(s_main, axis=1, keepdims=True),
                                    jnp.max(s_ex, axis=1, keepdims=True))
                    w1 = jnp.exp(s_main - m)
                    w2 = jnp.exp(s_ex - m)
                    den = (jnp.sum(w1, axis=1, keepdims=True)
                           + jnp.sum(w2, axis=1, keepdims=True))
                    wb1 = (w1 / den).astype(jnp.bfloat16)
                    wb2 = (w2 / den).astype(jnp.bfloat16)
                    ctx = jax.lax.dot_general(
                        wb1, v_bf[h, c0:c0 + cw, :], (((1,), (0,)), ((), ())),
                        preferred_element_type=jnp.float32)
                    ctx += jax.lax.dot_general(
                        wb2, v_ex, (((1,), (0,)), ((), ())),
                        preferred_element_type=jnp.float32)
                ctx_bf[roff:roff + m_rows,
                       h * DH:(h + 1) * DH] = ctx.astype(jnp.bfloat16)

        wo_b = wo_ref[...].astype(jnp.bfloat16)
        pf = jax.lax.dot_general(
            ctx_bf[...], wo_b, (((1,), (0,)), ((), ())),
            preferred_element_type=jnp.float32)
        for sp, qb in enumerate(SROW):
            p_ref[qb // 4, (qb % 4) * BLK:(qb % 4 + 1) * BLK, :] = (
                pf[sp * BLK:(sp + 1) * BLK])

        acc = None
        for t in range(N_DEV - 1):
            cs = lax.rem(my - 1 - t + 2 * N_DEV, N_DEV)
            cr = lax.rem(my - 2 - t + 2 * N_DEV, N_DEV)
            if t == 0:
                sendv = p_ref[pl.ds(cs, 1)].reshape(SQ_SHARD, D_MODEL)
            else:
                sendv = acc
            rs_send[t] = sendv.astype(jnp.bfloat16)
            rdma = pltpu.make_async_remote_copy(
                src_ref=rs_send.at[t],
                dst_ref=rs_recv.at[t],
                send_sem=rs_ssem.at[t],
                recv_sem=rs_rsem.at[t],
                device_id=(right,),
                device_id_type=pl.DeviceIdType.MESH,
            )
            rdma.start()
            rdma.wait()
            ploc = p_ref[pl.ds(cr, 1)].reshape(SQ_SHARD, D_MODEL)
            acc = rs_recv[t].astype(jnp.float32) + ploc
        out_ref[0] = acc

    out_shape = jax.ShapeDtypeStruct((1, SQ_SHARD, D_MODEL), jnp.float32)
    return pl.pallas_call(
        body,
        out_shape=out_shape,
        in_specs=[
            pl.BlockSpec(memory_space=pltpu.VMEM),
            pl.BlockSpec(memory_space=pltpu.VMEM),
            pl.BlockSpec(memory_space=pl.ANY),
            pl.BlockSpec(memory_space=pl.ANY),
            pl.BlockSpec(memory_space=pltpu.VMEM),
        ],
        out_specs=pl.BlockSpec(memory_space=pltpu.VMEM),
        scratch_shapes=[
            pltpu.VMEM((N_DEV, SQ_SHARD, D_MODEL), jnp.bfloat16),
            pltpu.VMEM((SQ, D_LOCAL), jnp.bfloat16),
            pltpu.VMEM((HQ_LOCAL, SKV, DH), jnp.bfloat16),
            pltpu.VMEM((HQ_LOCAL, SKV, DH), jnp.bfloat16),
            pltpu.VMEM((2, SKV, DH), jnp.float32),
            pltpu.VMEM((2, SKV, DH), jnp.float32),
            pltpu.VMEM((N_DEV, SQ_SHARD, D_MODEL), jnp.float32),
            pltpu.VMEM((N_DEV - 1, SQ_SHARD, D_MODEL), jnp.bfloat16),
            pltpu.VMEM((N_DEV - 1, SQ_SHARD, D_MODEL), jnp.bfloat16),
            pltpu.VMEM((SQ, D_LOCAL), jnp.bfloat16),
            pltpu.SemaphoreType.DMA((N_DEV - 1,)),
            pltpu.SemaphoreType.DMA((N_DEV - 1,)),
            pltpu.SemaphoreType.DMA((N_DEV - 1,)),
            pltpu.SemaphoreType.DMA((N_DEV - 1,)),
            pltpu.SemaphoreType.DMA((2, 2)),
        ],
        compiler_params=pltpu.CompilerParams(
            vmem_limit_bytes=100 * 1024 * 1024),
    )(x, Wq, K_ext, V_ext, Wo)


# device time: 89817 ns/iter; 1.7731x vs baseline; 1.2005x over previous
import jax
import jax.numpy as jnp
from jax import lax
from jax.experimental import pallas as pl
from jax.experimental.pallas import tpu as pltpu

N_DEV = 4
SQ = 1024
SQ_SHARD = 256
SKV = 4096
HQ_LOCAL = 8
DH = 128
D_LOCAL = HQ_LOCAL * DH
D_MODEL = 1024
SCALE = 0.08838834764831843
BLK = 64
N_QB = SQ // BLK
N_KB = SKV // BLK

SORTED_KB = (list(range(0, N_KB, 3)) + list(range(1, N_KB, 3))
             + list(range(2, N_KB, 3)))
POS = {kb: j for j, kb in enumerate(SORTED_KB)}
CNT = [len(range(c, N_KB, 3)) for c in range(3)]
OFF = [0, CNT[0], CNT[0] + CNT[1]]

QBS_G = [[qb for qb in range(N_QB) if qb % 3 == g] for g in range(3)]
ROW_OFF = [0, len(QBS_G[0]) * BLK,
           (len(QBS_G[0]) + len(QBS_G[1])) * BLK]
SROW = QBS_G[0] + QBS_G[1] + QBS_G[2]
NEX = 1 + len(QBS_G[1])


def kernel(x, Wq, K_ext, V_ext, Wo):
    def body(x_ref, wq_ref, k_hbm, v_hbm, wo_ref, out_ref,
             x_all, q_bf, k_bf, v_bf, kst, vst, p_ref, rs_send, rs_recv,
             ctx_bf, ag_ssem, ag_rsem, rs_ssem, rs_rsem, ld_sem):
        my = lax.axis_index("i")
        right = lax.rem(my + 1, N_DEV)
        left = lax.rem(my - 1 + N_DEV, N_DEV)

        def start_head(h):
            s = h % 2
            hk = my * HQ_LOCAL + h
            ds = []
            for j, kb in enumerate(SORTED_KB):
                ck = pltpu.make_async_copy(
                    k_hbm.at[0, pl.ds(kb * BLK, BLK), hk, :],
                    kst.at[s, pl.ds(j * BLK, BLK), :],
                    ld_sem.at[s, 0])
                cv = pltpu.make_async_copy(
                    v_hbm.at[0, pl.ds(kb * BLK, BLK), hk, :],
                    vst.at[s, pl.ds(j * BLK, BLK), :],
                    ld_sem.at[s, 1])
                ck.start()
                cv.start()
                ds += [ck, cv]
            return ds

        def cast_head(h, ds):
            for d in ds:
                d.wait()
            s = h % 2
            k_bf[h] = kst[s].astype(jnp.bfloat16)
            v_bf[h] = vst[s].astype(jnp.bfloat16)

        descs = {0: start_head(0), 1: start_head(1)}

        xv = x_ref[0].astype(jnp.bfloat16)
        x_all[pl.ds(my, 1)] = xv[None]
        d_r1 = pltpu.make_async_remote_copy(
            src_ref=x_all.at[my], dst_ref=x_all.at[my],
            send_sem=ag_ssem.at[0], recv_sem=ag_rsem.at[0],
            device_id=(right,), device_id_type=pl.DeviceIdType.MESH)
        d_l1 = pltpu.make_async_remote_copy(
            src_ref=x_all.at[my], dst_ref=x_all.at[my],
            send_sem=ag_ssem.at[1], recv_sem=ag_rsem.at[1],
            device_id=(left,), device_id_type=pl.DeviceIdType.MESH)
        d_r1.start()
        d_l1.start()
        cast_head(0, descs[0])
        d_r1.wait()
        sl2 = left
        d_r2 = pltpu.make_async_remote_copy(
            src_ref=x_all.at[sl2], dst_ref=x_all.at[sl2],
            send_sem=ag_ssem.at[2], recv_sem=ag_rsem.at[2],
            device_id=(right,), device_id_type=pl.DeviceIdType.MESH)
        d_r2.start()
        cast_head(1, descs[1])
        descs[2] = start_head(2)
        descs[3] = start_head(3)
        d_l1.wait()
        d_r2.wait()

        xg = x_all[...].reshape(SQ, D_MODEL)
        xs = jnp.concatenate(
            [xg[qb * BLK:(qb + 1) * BLK] for qb in SROW], axis=0)
        wq_b = wq_ref[...].astype(jnp.bfloat16)
        for half in range(2):
            qf = jax.lax.dot_general(
                xs[half * 512:(half + 1) * 512], wq_b,
                (((1,), (0,)), ((), ())),
                preferred_element_type=jnp.float32)
            q_bf[half * 512:(half + 1) * 512] = qf.astype(jnp.bfloat16)

        nr1 = ROW_OFF[2] - ROW_OFF[1]
        rb = lax.broadcasted_iota(jnp.int32, (nr1, NEX * BLK), 0) // BLK
        cb = lax.broadcasted_iota(jnp.int32, (nr1, NEX * BLK), 1) // BLK
        bias_ex = jnp.where((cb == 0) | (rb == cb - 1), 0.0,
                            -1e9).astype(jnp.float32)

        def attention(h):
            for g in range(3):
                r = (3 - g) % 3
                roff = ROW_OFF[g]
                m_rows = len(QBS_G[g]) * BLK
                c0 = OFF[r] * BLK
                cw = CNT[r] * BLK
                q = q_bf[roff:roff + m_rows, h * DH:(h + 1) * DH]
                s_main = jax.lax.dot_general(
                    q, k_bf[h, c0:c0 + cw, :], (((1,), (1,)), ((), ())),
                    preferred_element_type=jnp.float32) * SCALE
                if g == 0:
                    m = jnp.max(s_main, axis=1, keepdims=True)
                    w = jnp.exp(s_main - m)
                    den = jnp.sum(w, axis=1, keepdims=True)
                    wb = (w / den).astype(jnp.bfloat16)
                    ctx = jax.lax.dot_general(
                        wb, v_bf[h, c0:c0 + cw, :], (((1,), (0,)), ((), ())),
                        preferred_element_type=jnp.float32)
                else:
                    k_ex = jnp.concatenate(
                        [k_bf[h, 0:BLK, :]]
                        + [k_bf[h, POS[qb] * BLK:(POS[qb] + 1) * BLK, :]
                           for qb in QBS_G[g]], axis=0)
                    v_ex = jnp.concatenate(
                        [v_bf[h, 0:BLK, :]]
                        + [v_bf[h, POS[qb] * BLK:(POS[qb] + 1) * BLK, :]
                           for qb in QBS_G[g]], axis=0)
                    s_ex = jax.lax.dot_general(
                        q, k_ex, (((1,), (1,)), ((), ())),
                        preferred_element_type=jnp.float32) * SCALE + bias_ex
                    m = jnp.maximum(jnp.max(s_main, axis=1, keepdims=True),
                                    jnp.max(s_ex, axis=1, keepdims=True))
                    w1 = jnp.exp(s_main - m)
                    w2 = jnp.exp(s_ex - m)
                    den = (jnp.sum(w1, axis=1, keepdims=True)
                           + jnp.sum(w2, axis=1, keepdims=True))
                    wb1 = (w1 / den).astype(jnp.bfloat16)
                    wb2 = (w2 / den).astype(jnp.bfloat16)
                    ctx = jax.lax.dot_general(
                        wb1, v_bf[h, c0:c0 + cw, :], (((1,), (0,)), ((), ())),
                        preferred_element_type=jnp.float32)
                    ctx += jax.lax.dot_general(
                        wb2, v_ex, (((1,), (0,)), ((), ())),
                        preferred_element_type=jnp.float32)
                ctx_bf[roff:roff + m_rows,
                       h * DH:(h + 1) * DH] = ctx.astype(jnp.bfloat16)

        for h in range(2, HQ_LOCAL):
            cast_head(h, descs[h])
            if h + 2 < HQ_LOCAL:
                descs[h + 2] = start_head(h + 2)
            attention(h - 2)
        attention(HQ_LOCAL - 2)
        attention(HQ_LOCAL - 1)

        wo_b = wo_ref[...].astype(jnp.bfloat16)
        pf = jax.lax.dot_general(
            ctx_bf[...], wo_b, (((1,), (0,)), ((), ())),
            preferred_element_type=jnp.float32)
        for sp, qb in enumerate(SROW):
            p_ref[qb // 4, (qb % 4) * BLK:(qb % 4 + 1) * BLK, :] = (
                pf[sp * BLK:(sp + 1) * BLK])

        cA = lax.rem(my - 2 + N_DEV, N_DEV)
        cB = lax.rem(my + 1, N_DEV)
        cC = lax.rem(my - 1 + N_DEV, N_DEV)
        rs_send[0] = p_ref[pl.ds(cA, 1)].reshape(
            SQ_SHARD, D_MODEL).astype(jnp.bfloat16)
        rs_send[1] = p_ref[pl.ds(cB, 1)].reshape(
            SQ_SHARD, D_MODEL).astype(jnp.bfloat16)
        d_a = pltpu.make_async_remote_copy(
            src_ref=rs_send.at[0], dst_ref=rs_recv.at[0],
            send_sem=rs_ssem.at[0], recv_sem=rs_rsem.at[0],
            device_id=(left,), device_id_type=pl.DeviceIdType.MESH)
        d_b = pltpu.make_async_remote_copy(
            src_ref=rs_send.at[1], dst_ref=rs_recv.at[1],
            send_sem=rs_ssem.at[1], recv_sem=rs_rsem.at[1],
            device_id=(right,), device_id_type=pl.DeviceIdType.MESH)
        d_a.start()
        d_b.start()
        d_a.wait()
        comb = (rs_recv[0].astype(jnp.float32)
                + p_ref[pl.ds(cC, 1)].reshape(SQ_SHARD, D_MODEL))
        rs_send[2] = comb.astype(jnp.bfloat16)
        d_c = pltpu.make_async_remote_copy(
            src_ref=rs_send.at[2], dst_ref=rs_recv.at[2],
            send_sem=rs_ssem.at[2], recv_sem=rs_rsem.at[2],
            device_id=(left,), device_id_type=pl.DeviceIdType.MESH)
        d_c.start()
        d_b.wait()
        d_c.wait()
        out_ref[0] = (p_ref[pl.ds(my, 1)].reshape(SQ_SHARD, D_MODEL)
                      + rs_recv[1].astype(jnp.float32)
                      + rs_recv[2].astype(jnp.float32))

    out_shape = jax.ShapeDtypeStruct((1, SQ_SHARD, D_MODEL), jnp.float32)
    return pl.pallas_call(
        body,
        out_shape=out_shape,
        in_specs=[
            pl.BlockSpec(memory_space=pltpu.VMEM),
            pl.BlockSpec(memory_space=pltpu.VMEM),
            pl.BlockSpec(memory_space=pl.ANY),
            pl.BlockSpec(memory_space=pl.ANY),
            pl.BlockSpec(memory_space=pltpu.VMEM),
        ],
        out_specs=pl.BlockSpec(memory_space=pltpu.VMEM),
        scratch_shapes=[
            pltpu.VMEM((N_DEV, SQ_SHARD, D_MODEL), jnp.bfloat16),
            pltpu.VMEM((SQ, D_LOCAL), jnp.bfloat16),
            pltpu.VMEM((HQ_LOCAL, SKV, DH), jnp.bfloat16),
            pltpu.VMEM((HQ_LOCAL, SKV, DH), jnp.bfloat16),
            pltpu.VMEM((2, SKV, DH), jnp.float32),
            pltpu.VMEM((2, SKV, DH), jnp.float32),
            pltpu.VMEM((N_DEV, SQ_SHARD, D_MODEL), jnp.float32),
            pltpu.VMEM((N_DEV - 1, SQ_SHARD, D_MODEL), jnp.bfloat16),
            pltpu.VMEM((N_DEV - 1, SQ_SHARD, D_MODEL), jnp.bfloat16),
            pltpu.VMEM((SQ, D_LOCAL), jnp.bfloat16),
            pltpu.SemaphoreType.DMA((3,)),
            pltpu.SemaphoreType.DMA((3,)),
            pltpu.SemaphoreType.DMA((3,)),
            pltpu.SemaphoreType.DMA((3,)),
            pltpu.SemaphoreType.DMA((2, 2)),
        ],
        compiler_params=pltpu.CompilerParams(
            vmem_limit_bytes=100 * 1024 * 1024),
    )(x, Wq, K_ext, V_ext, Wo)
